# Initial kernel scaffold; baseline (speedup 1.0000x reference)
import jax
import jax.numpy as jnp
from jax.experimental import pallas as pl


def kernel(x, anchor_boxes):
    raise NotImplementedError("write your pallas kernel here")


# speedup vs baseline: 10.0685x; 10.0685x over previous
import jax
import jax.numpy as jnp
from jax.experimental import pallas as pl
from jax.experimental.pallas import tpu as pltpu

_TOP_N = 1000
_MAX_BOXES = 18
_NMS_THRESH = 0.5
_N_ANCHORS = 21600
_N_PAD = 21632
_B = 8
_INT_MIN = -(2**31)
_BIG = 2**31 - 1


def _nms_body(score_ref, dx_ref, dy_ref, dw_ref, dh_ref,
              xa_ref, ya_ref, wa_ref, ha_ref,
              ocx_ref, ocy_ref, ow_ref, oh_ref, os_ref,
              skey_ref, x1_ref, y1_ref, x2_ref, y2_ref, area_ref,
              cx_ref, cy_ref, w_ref, h_ref, supp_ref, pick_ref):
    score = score_ref[...]
    col = jax.lax.broadcasted_iota(jnp.int32, (_B, _N_PAD), 1)
    valid = col < _N_ANCHORS

    bits = jax.lax.bitcast_convert_type(score, jnp.int32)
    skey = jnp.where(bits >= 0, bits, bits ^ jnp.int32(0x7FFFFFFF))
    skey = jnp.where(valid, skey, _INT_MIN)
    skey_ref[...] = skey

    xa = xa_ref[...]
    ya = ya_ref[...]
    wa = wa_ref[...]
    ha = ha_ref[...]
    cx = dx_ref[...] * wa + xa
    cy = dy_ref[...] * ha + ya
    w = wa * jnp.exp(dw_ref[...])
    h = ha * jnp.exp(dh_ref[...])
    x1 = cx - w / 2.0
    y1 = cy - h / 2.0
    x2 = cx + w / 2.0
    y2 = cy + h / 2.0
    cx_ref[...] = cx
    cy_ref[...] = cy
    w_ref[...] = w
    h_ref[...] = h
    x1_ref[...] = x1
    y1_ref[...] = y1
    x2_ref[...] = x2
    y2_ref[...] = y2
    area_ref[...] = (x2 - x1) * (y2 - y1)

    def tstep(i, t):
        b = 31 - i
        cand = t + (jnp.int32(1) << b)
        cnt = jnp.sum((skey_ref[...] >= cand).astype(jnp.int32),
                      axis=1, keepdims=True)
        return jnp.where(cnt >= _TOP_N, cand, t)

    thr = jax.lax.fori_loop(0, 32, tstep,
                            jnp.full((_B, 1), _INT_MIN, jnp.int32))

    c_gt = jnp.sum((skey >= thr).astype(jnp.int32), axis=1, keepdims=True) \
        - jnp.sum((skey == thr).astype(jnp.int32), axis=1, keepdims=True)
    need = _TOP_N - c_gt

    def istep(i, m):
        b = 14 - i
        test = m + (jnp.int32(1) << b) - 1
        sk = skey_ref[...]
        cnt = jnp.sum(((sk == thr) & (col <= test)).astype(jnp.int32),
                      axis=1, keepdims=True)
        return jnp.where(cnt < need, m + (jnp.int32(1) << b), m)

    mcut = jax.lax.fori_loop(0, 15, istep, jnp.zeros((_B, 1), jnp.int32))
    not_cand = ((skey < thr) | ((skey == thr) & (col > mcut)))
    supp_ref[...] = not_cand.astype(jnp.int32)
    pick_ref[...] = not_cand.astype(jnp.int32)

    iota18 = jax.lax.broadcasted_iota(jnp.int32, (_B, _MAX_BOXES), 1)
    zeros18 = jnp.zeros((_B, _MAX_BOXES), jnp.float32)

    def gather_at(onehot, arr):
        return jnp.sum(jnp.where(onehot, arr, 0.0), axis=1, keepdims=True)

    def pick_step(t, carry):
        kcx, kcy, kw, kh, ks, nk = carry
        sk = skey_ref[...]
        avail = supp_ref[...] == 0
        mkey = jnp.where(avail, sk, _INT_MIN)
        mx = jnp.max(mkey, axis=1, keepdims=True)
        exists = mx > _INT_MIN
        pickm = avail & (sk == mx)
        j = jnp.min(jnp.where(pickm, col, _BIG), axis=1, keepdims=True)
        onehot = col == j
        x1v = x1_ref[...]
        y1v = y1_ref[...]
        x2v = x2_ref[...]
        y2v = y2_ref[...]
        gx1 = gather_at(onehot, x1v)
        gy1 = gather_at(onehot, y1v)
        gx2 = gather_at(onehot, x2v)
        gy2 = gather_at(onehot, y2v)
        xx1 = jnp.maximum(gx1, x1v)
        yy1 = jnp.maximum(gy1, y1v)
        xx2 = jnp.minimum(gx2, x2v)
        yy2 = jnp.minimum(gy2, y2v)
        ww = jnp.maximum(0.0, xx2 - xx1)
        hh = jnp.maximum(0.0, yy2 - yy1)
        ov = ww * hh / area_ref[...]
        newsupp = exists & ((ov > _NMS_THRESH) | onehot)
        supp_ref[...] = supp_ref[...] | newsupp.astype(jnp.int32)
        slotm = (iota18 == t) & exists
        kcx = jnp.where(slotm, gather_at(onehot, cx_ref[...]), kcx)
        kcy = jnp.where(slotm, gather_at(onehot, cy_ref[...]), kcy)
        kw = jnp.where(slotm, gather_at(onehot, w_ref[...]), kw)
        kh = jnp.where(slotm, gather_at(onehot, h_ref[...]), kh)
        ks = jnp.where(slotm, gather_at(onehot, score_ref[...]), ks)
        nk = nk + exists.astype(jnp.int32)
        return kcx, kcy, kw, kh, ks, nk

    init = (zeros18, zeros18, zeros18, zeros18, zeros18,
            jnp.zeros((_B, 1), jnp.int32))
    kcx, kcy, kw, kh, ks, nk = jax.lax.fori_loop(
        0, _MAX_BOXES, pick_step, init)

    def rank_step(r, carry):
        ocx, ocy, ow, oh, osc = carry
        sk = skey_ref[...]
        avail = pick_ref[...] == 0
        mkey = jnp.where(avail, sk, _INT_MIN)
        mx = jnp.max(mkey, axis=1, keepdims=True)
        pickm = avail & (sk == mx)
        j = jnp.min(jnp.where(pickm, col, _BIG), axis=1, keepdims=True)
        onehot = col == j
        pick_ref[...] = pick_ref[...] | onehot.astype(jnp.int32)
        slotm = iota18 == (nk + r)
        ocx = jnp.where(slotm, gather_at(onehot, cx_ref[...]), ocx)
        ocy = jnp.where(slotm, gather_at(onehot, cy_ref[...]), ocy)
        ow = jnp.where(slotm, gather_at(onehot, w_ref[...]), ow)
        oh = jnp.where(slotm, gather_at(onehot, h_ref[...]), oh)
        osc = jnp.where(slotm, gather_at(onehot, score_ref[...]), osc)
        return ocx, ocy, ow, oh, osc

    keptm = iota18 < nk
    rinit = (jnp.where(keptm, kcx, 0.0), jnp.where(keptm, kcy, 0.0),
             jnp.where(keptm, kw, 0.0), jnp.where(keptm, kh, 0.0),
             jnp.where(keptm, ks, 0.0))
    ocx, ocy, ow, oh, osc = jax.lax.fori_loop(
        0, _MAX_BOXES, rank_step, rinit)

    ocx_ref[...] = ocx
    ocy_ref[...] = ocy
    ow_ref[...] = ow
    oh_ref[...] = oh
    os_ref[...] = osc


def _pallas_nms(score, dx, dy, dw, dh, xa, ya, wa, ha):
    out_sds = [jax.ShapeDtypeStruct((_B, _MAX_BOXES), jnp.float32)] * 5
    fscratch = [pltpu.VMEM((_B, _N_PAD), jnp.float32)] * 9
    iscratch = [pltpu.VMEM((_B, _N_PAD), jnp.int32)] * 3
    return pl.pallas_call(
        _nms_body,
        out_shape=out_sds,
        scratch_shapes=[iscratch[0]] + fscratch + iscratch[1:],
    )(score, dx, dy, dw, dh, xa, ya, wa, ha)


def kernel(x, anchor_boxes):
    npad = _N_PAD - _N_ANCHORS

    def pad_x(a):
        return jnp.pad(a, ((0, 0), (0, npad)))

    score = pad_x(x[:, :, 0])
    dx = pad_x(x[:, :, 2])
    dy = pad_x(x[:, :, 3])
    dw = pad_x(x[:, :, 4])
    dh = pad_x(x[:, :, 5])
    anc = jnp.transpose(anchor_boxes, (1, 0, 2, 3)).reshape(_N_ANCHORS, 4)

    def pad_a(a):
        return jnp.pad(a, (0, npad)).reshape(1, _N_PAD)

    xa = pad_a(anc[:, 0])
    ya = pad_a(anc[:, 1])
    wa = pad_a(anc[:, 2])
    ha = pad_a(anc[:, 3])
    ocx, ocy, ow, oh, osc = _pallas_nms(score, dx, dy, dw, dh, xa, ya, wa, ha)
    return jnp.stack([ocx, ocy, ow, oh, osc], axis=-1)


# speedup vs baseline: 12.4393x; 1.2355x over previous
import jax
import jax.numpy as jnp
from jax.experimental import pallas as pl
from jax.experimental.pallas import tpu as pltpu

_TOP_N = 1000
_MAX_BOXES = 18
_NMS_THRESH = 0.5
_N_ANCHORS = 21600
_N_PAD = 21632
_B = 8
_INT_MIN = -(2**31)
_BIG = 2**31 - 1


def _nms_body(score_ref, dx_ref, dy_ref, dw_ref, dh_ref,
              xa_ref, ya_ref, wa_ref, ha_ref,
              ocx_ref, ocy_ref, ow_ref, oh_ref, os_ref,
              skey_ref, x1_ref, y1_ref, x2_ref, y2_ref, area_ref,
              supp_ref, pick_ref):
    score = score_ref[...]
    col = jax.lax.broadcasted_iota(jnp.int32, (_B, _N_PAD), 1)
    valid = col < _N_ANCHORS

    bits = jax.lax.bitcast_convert_type(score, jnp.int32)
    skey = jnp.where(bits >= 0, bits, bits ^ jnp.int32(0x7FFFFFFF))
    skey = jnp.where(valid, skey, _INT_MIN)
    skey_ref[...] = skey

    xa = xa_ref[...]
    ya = ya_ref[...]
    wa = wa_ref[...]
    ha = ha_ref[...]
    cx = dx_ref[...] * wa + xa
    cy = dy_ref[...] * ha + ya
    w = wa * jnp.exp(dw_ref[...])
    h = ha * jnp.exp(dh_ref[...])
    x1 = cx - w / 2.0
    y1 = cy - h / 2.0
    x2 = cx + w / 2.0
    y2 = cy + h / 2.0
    x1_ref[...] = x1
    y1_ref[...] = y1
    x2_ref[...] = x2
    y2_ref[...] = y2
    area_ref[...] = (x2 - x1) * (y2 - y1)

    def tstep(i, t):
        b = 31 - i
        cand = t + (jnp.int32(1) << b)
        cnt = jnp.sum((skey_ref[...] >= cand).astype(jnp.int32),
                      axis=1, keepdims=True)
        return jnp.where(cnt >= _TOP_N, cand, t)

    thr = jax.lax.fori_loop(0, 32, tstep,
                            jnp.full((_B, 1), _INT_MIN, jnp.int32))

    c_gt = jnp.sum((skey > thr).astype(jnp.int32), axis=1, keepdims=True)
    need = _TOP_N - c_gt

    def istep(i, m):
        b = 14 - i
        test = m + (jnp.int32(1) << b) - 1
        sk = skey_ref[...]
        cnt = jnp.sum(((sk == thr) & (col <= test)).astype(jnp.int32),
                      axis=1, keepdims=True)
        return jnp.where(cnt < need, m + (jnp.int32(1) << b), m)

    mcut = jax.lax.fori_loop(0, 15, istep, jnp.zeros((_B, 1), jnp.int32))
    not_cand = ((skey < thr) | ((skey == thr) & (col > mcut)))
    supp_ref[...] = not_cand.astype(jnp.int32)
    pick_ref[...] = not_cand.astype(jnp.int32)

    iota18 = jax.lax.broadcasted_iota(jnp.int32, (_B, _MAX_BOXES), 1)
    zeros18 = jnp.zeros((_B, _MAX_BOXES), jnp.float32)

    def gather_at(onehot, arr):
        return jnp.sum(jnp.where(onehot, arr, 0.0), axis=1, keepdims=True)

    def pick_step(t, carry):
        kx1, ky1, kx2, ky2, ks, nk = carry
        sk = skey_ref[...]
        avail = supp_ref[...] == 0
        mkey = jnp.where(avail, sk, _INT_MIN)
        mx = jnp.max(mkey, axis=1, keepdims=True)
        exists = mx > _INT_MIN
        pickm = avail & (sk == mx)
        j = jnp.min(jnp.where(pickm, col, _BIG), axis=1, keepdims=True)
        onehot = col == j
        x1v = x1_ref[...]
        y1v = y1_ref[...]
        x2v = x2_ref[...]
        y2v = y2_ref[...]
        gx1 = gather_at(onehot, x1v)
        gy1 = gather_at(onehot, y1v)
        gx2 = gather_at(onehot, x2v)
        gy2 = gather_at(onehot, y2v)
        gs = gather_at(onehot, score_ref[...])
        xx1 = jnp.maximum(gx1, x1v)
        yy1 = jnp.maximum(gy1, y1v)
        xx2 = jnp.minimum(gx2, x2v)
        yy2 = jnp.minimum(gy2, y2v)
        ww = jnp.maximum(0.0, xx2 - xx1)
        hh = jnp.maximum(0.0, yy2 - yy1)
        ov = ww * hh / area_ref[...]
        newsupp = exists & ((ov > _NMS_THRESH) | onehot)
        supp_ref[...] = supp_ref[...] | newsupp.astype(jnp.int32)
        slotm = (iota18 == t) & exists
        kx1 = jnp.where(slotm, gx1, kx1)
        ky1 = jnp.where(slotm, gy1, ky1)
        kx2 = jnp.where(slotm, gx2, kx2)
        ky2 = jnp.where(slotm, gy2, ky2)
        ks = jnp.where(slotm, gs, ks)
        nk = nk + exists.astype(jnp.int32)
        return kx1, ky1, kx2, ky2, ks, nk

    init = (zeros18, zeros18, zeros18, zeros18, zeros18,
            jnp.zeros((_B, 1), jnp.int32))
    kx1, ky1, kx2, ky2, ks, nk = jax.lax.fori_loop(
        0, _MAX_BOXES, pick_step, init)

    def rank_step(r, carry):
        px1, py1, px2, py2, psc = carry
        sk = skey_ref[...]
        avail = pick_ref[...] == 0
        mkey = jnp.where(avail, sk, _INT_MIN)
        mx = jnp.max(mkey, axis=1, keepdims=True)
        pickm = avail & (sk == mx)
        j = jnp.min(jnp.where(pickm, col, _BIG), axis=1, keepdims=True)
        onehot = col == j
        pick_ref[...] = pick_ref[...] | onehot.astype(jnp.int32)
        slotm = iota18 == (nk + r)
        px1 = jnp.where(slotm, gather_at(onehot, x1_ref[...]), px1)
        py1 = jnp.where(slotm, gather_at(onehot, y1_ref[...]), py1)
        px2 = jnp.where(slotm, gather_at(onehot, x2_ref[...]), px2)
        py2 = jnp.where(slotm, gather_at(onehot, y2_ref[...]), py2)
        psc = jnp.where(slotm, gather_at(onehot, score_ref[...]), psc)
        return px1, py1, px2, py2, psc

    keptm = iota18 < nk
    rinit = (jnp.where(keptm, kx1, 0.0), jnp.where(keptm, ky1, 0.0),
             jnp.where(keptm, kx2, 0.0), jnp.where(keptm, ky2, 0.0),
             jnp.where(keptm, ks, 0.0))
    n_pad_slots = _MAX_BOXES - jnp.min(nk)
    fx1, fy1, fx2, fy2, fsc = jax.lax.fori_loop(
        0, n_pad_slots, rank_step, rinit)

    ocx_ref[...] = (fx1 + fx2) * 0.5
    ocy_ref[...] = (fy1 + fy2) * 0.5
    ow_ref[...] = fx2 - fx1
    oh_ref[...] = fy2 - fy1
    os_ref[...] = fsc


def _pallas_nms(score, dx, dy, dw, dh, xa, ya, wa, ha):
    out_sds = [jax.ShapeDtypeStruct((_B, _MAX_BOXES), jnp.float32)] * 5
    f32v = lambda: pltpu.VMEM((_B, _N_PAD), jnp.float32)
    i32v = lambda: pltpu.VMEM((_B, _N_PAD), jnp.int32)
    return pl.pallas_call(
        _nms_body,
        out_shape=out_sds,
        scratch_shapes=[i32v(), f32v(), f32v(), f32v(), f32v(), f32v(),
                        i32v(), i32v()],
    )(score, dx, dy, dw, dh, xa, ya, wa, ha)


def kernel(x, anchor_boxes):
    npad = _N_PAD - _N_ANCHORS

    def pad_x(a):
        return jnp.pad(a, ((0, 0), (0, npad)))

    score = pad_x(x[:, :, 0])
    dx = pad_x(x[:, :, 2])
    dy = pad_x(x[:, :, 3])
    dw = pad_x(x[:, :, 4])
    dh = pad_x(x[:, :, 5])
    anc = jnp.transpose(anchor_boxes, (1, 0, 2, 3)).reshape(_N_ANCHORS, 4)

    def pad_a(a):
        return jnp.pad(a, (0, npad)).reshape(1, _N_PAD)

    xa = pad_a(anc[:, 0])
    ya = pad_a(anc[:, 1])
    wa = pad_a(anc[:, 2])
    ha = pad_a(anc[:, 3])
    ocx, ocy, ow, oh, osc = _pallas_nms(score, dx, dy, dw, dh, xa, ya, wa, ha)
    return jnp.stack([ocx, ocy, ow, oh, osc], axis=-1)
